# baseline (speedup 1.0000x reference)
import jax
import jax.numpy as jnp
from jax import lax
from jax.experimental import pallas as pl
from jax.experimental.pallas import tpu as pltpu
from jax.experimental.pallas import tpu_sc as plsc

_B = 4096
_D = 128
_K = 2
_N = 32

_R = _B * _K
_NW = 32
_RPW = _R // _NW
_CH = 8
_NCHUNK = _RPW // _CH

_BB = 256


def _sc_pool_body(nbr_hbm, m_hbm, out_hbm,
                  buf0, buf1, mball, ob0, ob1,
                  sem0, sem1, msem, osem0, osem1):
    c = lax.axis_index("c")
    s = lax.axis_index("s")
    wid = s * 2 + c
    row0 = wid * _RPW
    bufs = [buf0, buf1]
    sems = [sem0, sem1]
    obs = [ob0, ob1]
    osems = [osem0, osem1]

    pltpu.async_copy(m_hbm.at[pl.ds(row0, _RPW)], mball, msem)

    def issue(g, b):
        pltpu.async_copy(nbr_hbm.at[pl.ds(row0 + g * _CH, _CH)], bufs[b], sems[b])

    def wait_in(b):
        pltpu.make_async_copy(nbr_hbm.at[pl.ds(0, _CH)], bufs[b], sems[b]).wait()

    issue(0, 0)
    pltpu.make_async_copy(m_hbm.at[pl.ds(0, _RPW)], mball, msem).wait()

    def pair(p, carry):
        for b in range(2):
            g = 2 * p + b
            wait_in(b)

            @pl.when(g + 1 < _NCHUNK)
            def _():
                issue(g + 1, 1 - b)

            @pl.when(p >= 1)
            def _():
                pltpu.make_async_copy(obs[b], out_hbm.at[pl.ds(0, _CH)],
                                      osems[b]).wait()

            buf = bufs[b]
            obuf = obs[b]

            def row(i, carry2):
                acc = [jnp.zeros((16,), jnp.float32) for _ in range(8)]
                mrow = g * _CH + i
                mv0 = mball[mrow, pl.ds(0, 16)]
                mv1 = mball[mrow, pl.ds(16, 16)]
                for n in range(_N):
                    mn = mv0[n] if n < 16 else mv1[n - 16]
                    for j in range(8):
                        acc[j] = acc[j] + mn * buf[i, n, pl.ds(j * 16, 16)]
                for j in range(8):
                    obuf[i, pl.ds(j * 16, 16)] = acc[j]
                return carry2

            lax.fori_loop(0, _CH, row, 0, unroll=2)
            pltpu.async_copy(obuf, out_hbm.at[pl.ds(row0 + g * _CH, _CH)],
                             osems[b])
        return carry

    lax.fori_loop(0, _NCHUNK // 2, pair, 0)
    pltpu.make_async_copy(ob0, out_hbm.at[pl.ds(0, _CH)], osem0).wait()
    pltpu.make_async_copy(ob1, out_hbm.at[pl.ds(0, _CH)], osem1).wait()


def _sc_pool(nbr3, m2):
    mesh = plsc.VectorSubcoreMesh(core_axis_name="c", subcore_axis_name="s")
    f = pl.kernel(
        _sc_pool_body,
        mesh=mesh,
        out_type=jax.ShapeDtypeStruct((_R, _D), jnp.float32),
        scratch_types=[
            pltpu.VMEM((_CH, _N, _D), jnp.float32),
            pltpu.VMEM((_CH, _N, _D), jnp.float32),
            pltpu.VMEM((_RPW, _N), jnp.float32),
            pltpu.VMEM((_CH, _D), jnp.float32),
            pltpu.VMEM((_CH, _D), jnp.float32),
            pltpu.SemaphoreType.DMA,
            pltpu.SemaphoreType.DMA,
            pltpu.SemaphoreType.DMA,
            pltpu.SemaphoreType.DMA,
            pltpu.SemaphoreType.DMA,
        ],
    )
    return f(nbr3, m2)


def _mm_body(e_ref, sv_ref, wt_ref, b_ref, out_ref):
    scale = jnp.float32(1.0 / _N)
    x0 = sv_ref[...]
    e0 = e_ref[:, 0, :] * scale
    e1 = e_ref[:, 1, :] * scale
    acc = jnp.dot(x0, wt_ref[0:_D, :], preferred_element_type=jnp.float32)
    acc += jnp.dot(e0, wt_ref[_D:2 * _D, :], preferred_element_type=jnp.float32)
    acc += jnp.dot(e1, wt_ref[2 * _D:3 * _D, :], preferred_element_type=jnp.float32)
    out_ref[...] = acc + b_ref[...]


def _tc_matmul(e, sv, wt, bb):
    grid = (_B // _BB,)
    return pl.pallas_call(
        _mm_body,
        grid=grid,
        in_specs=[
            pl.BlockSpec((_BB, _K, _D), lambda i: (i, 0, 0)),
            pl.BlockSpec((_BB, _D), lambda i: (i, 0)),
            pl.BlockSpec((3 * _D, _D), lambda i: (0, 0)),
            pl.BlockSpec((1, _D), lambda i: (0, 0)),
        ],
        out_specs=pl.BlockSpec((_BB, _D), lambda i: (i, 0)),
        out_shape=jax.ShapeDtypeStruct((_B, _D), jnp.float32),
        compiler_params=pltpu.CompilerParams(
            dimension_semantics=("arbitrary",),
        ),
    )(e, sv, wt, bb)


def kernel(self_vectors, neighbor_vectors, masks, W, b):
    nbr3 = neighbor_vectors.reshape(_R, _N, _D)
    m2 = masks.reshape(_R, _N)
    sv = self_vectors.reshape(_B, _D)
    wt = W.T
    bb = b.reshape(1, _D)

    e_sc = _sc_pool(nbr3, m2)
    out = _tc_matmul(e_sc.reshape(_B, _K, _D), sv, wt, bb)
    return out.reshape(_B, 1, _D)
